# baseline (speedup 1.0000x reference)
import jax
import jax.numpy as jnp
from jax import lax
from jax.experimental import pallas as pl
from jax.experimental.pallas import tpu as pltpu
from jax.experimental.pallas import tpu_sc as plsc

N_ATOMS = 1638400
N_MOL = 16384
W_FORCE = 0.999
W_ENERGY = 0.001

NC = 2
NS = 16
LANES = 16

NW = NC * NS
APW = N_ATOMS // NW
APL = APW // LANES
ROUNDS = 5
CPL = APL // ROUNDS
GPR = CPL // 128
MPT = N_MOL // NS


def _sc_body(d_hbm, idx_hbm, out_hbm,
             d_buf, i_buf, acc, red, tmp, slab, sem):
    c = lax.axis_index("c")
    s = lax.axis_index("s")
    wid = c * NS + s
    wbase = wid * APW

    lanes = lax.iota(jnp.int32, LANES)
    zero16 = jnp.zeros((LANES,), jnp.float32)

    def _zero(j, carry):
        acc[pl.ds(j * LANES, LANES)] = zero16
        return carry
    lax.fori_loop(0, N_MOL // LANES, _zero, 0)

    DSZ = LANES * 3 * CPL
    ISZ = LANES * CPL
    lane_d = lanes * (3 * CPL)
    lane_i = lanes * CPL

    def _issue(r, b):
        descs = []
        for L in range(LANES):
            a0 = wbase + L * APL + r * CPL
            descs.append(pltpu.async_copy(
                d_hbm.at[pl.ds(3 * a0, 3 * CPL)],
                d_buf.at[pl.ds(b * DSZ + L * 3 * CPL, 3 * CPL)], sem))
            descs.append(pltpu.async_copy(
                idx_hbm.at[pl.ds(a0, CPL)],
                i_buf.at[pl.ds(b * ISZ + L * CPL, CPL)], sem))
        return descs

    def _gm(i32vec):
        return plsc.load_gather(i_buf, [i32vec])

    acc_v = zero16
    pend = _issue(0, 0)
    for r in range(ROUNDS):
        b = r % 2
        for d in pend:
            d.wait()
        pend = _issue(r + 1, (r + 1) % 2) if r + 1 < ROUNDS else []
        ib = b * ISZ + lane_i
        db = b * DSZ + lane_d

        def _body(i, acc_v, _ib=ib, _db=db):
            m = _gm(_ib + i)
            m_next = _gm(_ib + i + 1)
            g = jax.lax.shift_right_logical(i, 7)
            l = jax.lax.bitwise_and(i, 127)
            base = _db + (g * 384 + l)
            dx = plsc.load_gather(d_buf, [base])
            dy = plsc.load_gather(d_buf, [base + 128])
            dz = plsc.load_gather(d_buf, [base + 256])
            acc_v = acc_v + dx * dx + dy * dy + dz * dz
            boundary = m != m_next
            plsc.addupdate_scatter(acc, [m], acc_v, mask=boundary)
            return jnp.where(boundary, 0.0, acc_v)

        acc_v = plsc.parallel_loop(
            0, CPL - 1, unroll=16, carry=acc_v)(_body)

        i_last = CPL - 1
        m = _gm(ib + i_last)
        base = db + ((GPR - 1) * 384 + 127)
        dx = plsc.load_gather(d_buf, [base])
        dy = plsc.load_gather(d_buf, [base + 128])
        dz = plsc.load_gather(d_buf, [base + 256])
        acc_v = acc_v + dx * dx + dy * dy + dz * dz
        if r + 1 < ROUNDS:
            for d in pend:
                d.wait()
            pend = []
            m_next = _gm(((r + 1) % 2) * ISZ + lane_i)
            boundary = m != m_next
            plsc.addupdate_scatter(acc, [m], acc_v, mask=boundary)
            acc_v = jnp.where(boundary, 0.0, acc_v)
        else:
            m_last = m

    for k in range(LANES):
        plsc.addupdate_scatter(acc, [m_last], acc_v, mask=(lanes == k))

    pltpu.sync_copy(acc, slab.at[pl.ds(s * N_MOL, N_MOL)])
    plsc.subcore_barrier()
    pltpu.sync_copy(slab.at[pl.ds(s * MPT, MPT)], red)
    for j in range(1, NS):
        pltpu.sync_copy(slab.at[pl.ds(j * N_MOL + s * MPT, MPT)], tmp)

        def _acc(q, carry):
            sl = pl.ds(q * LANES, LANES)
            red[sl] = red[sl] + tmp[sl]
            return carry
        lax.fori_loop(0, MPT // LANES, _acc, 0)
    pltpu.sync_copy(red, out_hbm.at[c, pl.ds(s * MPT, MPT)])


_sc_partial = pl.kernel(
    _sc_body,
    out_type=jax.ShapeDtypeStruct((NC, N_MOL), jnp.float32),
    mesh=plsc.VectorSubcoreMesh(core_axis_name="c", subcore_axis_name="s"),
    scratch_types=[
        pltpu.VMEM((2 * LANES * 3 * CPL,), jnp.float32),
        pltpu.VMEM((2 * LANES * CPL,), jnp.int32),
        pltpu.VMEM((N_MOL,), jnp.float32),
        pltpu.VMEM((MPT,), jnp.float32),
        pltpu.VMEM((MPT,), jnp.float32),
        pltpu.VMEM_SHARED((NS * N_MOL,), jnp.float32),
        pltpu.SemaphoreType.DMA,
    ],
    compiler_params=pltpu.CompilerParams(needs_layout_passes=False),
)


def _fin_body(pm0, pm1, cnt, ep, et, tot, lf, le):
    counts = cnt[...].astype(jnp.float32)
    force = (pm0[...] + pm1[...]) / (3.0 * counts)
    d = ep[...] - et[...]
    energy = (d * d) / counts
    tot[...] = W_FORCE * force + W_ENERGY * energy
    lf[...] = force
    le[...] = energy


_R = 128


def kernel(per_atom_force_predict, per_atom_force_true,
           per_molecule_energy_predict, per_molecule_energy_true,
           atomic_subsystem_indices, atomic_subsystem_counts):
    d_flat = ((per_atom_force_predict - per_atom_force_true)
              .T.reshape(3, N_ATOMS // 128, 128)
              .transpose(1, 0, 2).reshape(-1))
    partial = _sc_partial(d_flat, atomic_subsystem_indices)

    shp = jax.ShapeDtypeStruct((_R, N_MOL // _R), jnp.float32)
    tot, lf, le = pl.pallas_call(
        _fin_body,
        out_shape=(shp, shp, shp),
    )(
        partial[0].reshape(_R, -1),
        partial[1].reshape(_R, -1),
        atomic_subsystem_counts.reshape(_R, -1),
        per_molecule_energy_predict.reshape(_R, -1),
        per_molecule_energy_true.reshape(_R, -1),
    )
    out = (tot.reshape(N_MOL, 1), lf.reshape(N_MOL, 1), le.reshape(N_MOL, 1))
    return out


# speedup vs baseline: 1.0097x; 1.0097x over previous
import jax
import jax.numpy as jnp
from jax import lax
from jax.experimental import pallas as pl
from jax.experimental.pallas import tpu as pltpu
from jax.experimental.pallas import tpu_sc as plsc

N_ATOMS = 1638400
N_MOL = 16384
W_FORCE = 0.999
W_ENERGY = 0.001

NC = 2
NS = 16
LANES = 16

NW = NC * NS
APW = N_ATOMS // NW
APL = APW // LANES
ROUNDS = 5
CPL = APL // ROUNDS
GPR = CPL // 128
MPT = N_MOL // NS


def _sc_body(d_hbm, idx_hbm, out_hbm,
             d_buf, i_buf, acc, red, tmp, slab, sem):
    c = lax.axis_index("c")
    s = lax.axis_index("s")
    wid = c * NS + s
    wbase = wid * APW

    lanes = lax.iota(jnp.int32, LANES)
    zero16 = jnp.zeros((LANES,), jnp.float32)

    def _zero(j, carry):
        acc[pl.ds(j * LANES, LANES)] = zero16
        return carry
    lax.fori_loop(0, N_MOL // LANES, _zero, 0)

    DSZ = LANES * 3 * CPL
    ISZ = LANES * CPL
    lane_d = lanes * (3 * CPL)
    lane_i = lanes * CPL

    def _issue(r, b):
        descs = []
        for L in range(LANES):
            a0 = wbase + L * APL + r * CPL
            descs.append(pltpu.async_copy(
                d_hbm.at[pl.ds(3 * a0, 3 * CPL)],
                d_buf.at[pl.ds(b * DSZ + L * 3 * CPL, 3 * CPL)], sem))
            descs.append(pltpu.async_copy(
                idx_hbm.at[pl.ds(a0, CPL)],
                i_buf.at[pl.ds(b * ISZ + L * CPL, CPL)], sem))
        return descs

    def _gm(i32vec):
        return plsc.load_gather(i_buf, [i32vec])

    acc_v = zero16
    pend = _issue(0, 0)
    for r in range(ROUNDS):
        b = r % 2
        for d in pend:
            d.wait()
        pend = _issue(r + 1, (r + 1) % 2) if r + 1 < ROUNDS else []
        ib = b * ISZ + lane_i
        db = b * DSZ + lane_d

        def _one(pos_d, acc_v, m, m_next):
            dx = plsc.load_gather(d_buf, [pos_d])
            dy = plsc.load_gather(d_buf, [pos_d + 128])
            dz = plsc.load_gather(d_buf, [pos_d + 256])
            acc_v = acc_v + dx * dx + dy * dy + dz * dz
            boundary = m != m_next
            plsc.addupdate_scatter(acc, [m], acc_v, mask=boundary)
            return jnp.where(boundary, 0.0, acc_v)

        def _block(i, acc_v, _ib=ib, _db=db):
            g = jax.lax.shift_right_logical(i, 7)
            si = _ib + i
            sd = _db + (i + g * 256)
            ms = [_gm(si + k) for k in range(9)]
            for k in range(8):
                acc_v = _one(sd + k, acc_v, ms[k], ms[k + 1])
            return acc_v

        acc_v = plsc.parallel_loop(
            0, CPL - 8, step=8, carry=acc_v)(_block)

        si = ib + (CPL - 8)
        sd = db + ((GPR - 1) * 384 + 120)
        ms = [_gm(si + k) for k in range(8)]
        for k in range(7):
            acc_v = _one(sd + k, acc_v, ms[k], ms[k + 1])
        m = ms[7]
        dx = plsc.load_gather(d_buf, [sd + 7])
        dy = plsc.load_gather(d_buf, [sd + 135])
        dz = plsc.load_gather(d_buf, [sd + 263])
        acc_v = acc_v + dx * dx + dy * dy + dz * dz
        if r + 1 < ROUNDS:
            for d in pend:
                d.wait()
            pend = []
            m_next = _gm(((r + 1) % 2) * ISZ + lane_i)
            boundary = m != m_next
            plsc.addupdate_scatter(acc, [m], acc_v, mask=boundary)
            acc_v = jnp.where(boundary, 0.0, acc_v)
        else:
            m_last = m

    for k in range(LANES):
        plsc.addupdate_scatter(acc, [m_last], acc_v, mask=(lanes == k))

    pltpu.sync_copy(acc, slab.at[pl.ds(s * N_MOL, N_MOL)])
    plsc.subcore_barrier()
    pltpu.sync_copy(slab.at[pl.ds(s * MPT, MPT)], red)
    for j in range(1, NS):
        pltpu.sync_copy(slab.at[pl.ds(j * N_MOL + s * MPT, MPT)], tmp)

        def _acc(q, carry):
            sl = pl.ds(q * LANES, LANES)
            red[sl] = red[sl] + tmp[sl]
            return carry
        lax.fori_loop(0, MPT // LANES, _acc, 0)
    pltpu.sync_copy(red, out_hbm.at[c, pl.ds(s * MPT, MPT)])


_sc_partial = pl.kernel(
    _sc_body,
    out_type=jax.ShapeDtypeStruct((NC, N_MOL), jnp.float32),
    mesh=plsc.VectorSubcoreMesh(core_axis_name="c", subcore_axis_name="s"),
    scratch_types=[
        pltpu.VMEM((2 * LANES * 3 * CPL,), jnp.float32),
        pltpu.VMEM((2 * LANES * CPL,), jnp.int32),
        pltpu.VMEM((N_MOL,), jnp.float32),
        pltpu.VMEM((MPT,), jnp.float32),
        pltpu.VMEM((MPT,), jnp.float32),
        pltpu.VMEM_SHARED((NS * N_MOL,), jnp.float32),
        pltpu.SemaphoreType.DMA,
    ],
    compiler_params=pltpu.CompilerParams(needs_layout_passes=False),
)


def _fin_body(pm0, pm1, cnt, ep, et, tot, lf, le):
    counts = cnt[...].astype(jnp.float32)
    force = (pm0[...] + pm1[...]) / (3.0 * counts)
    d = ep[...] - et[...]
    energy = (d * d) / counts
    tot[...] = W_FORCE * force + W_ENERGY * energy
    lf[...] = force
    le[...] = energy


_R = 128


def kernel(per_atom_force_predict, per_atom_force_true,
           per_molecule_energy_predict, per_molecule_energy_true,
           atomic_subsystem_indices, atomic_subsystem_counts):
    d_flat = ((per_atom_force_predict - per_atom_force_true)
              .T.reshape(3, N_ATOMS // 128, 128)
              .transpose(1, 0, 2).reshape(-1))
    partial = _sc_partial(d_flat, atomic_subsystem_indices)

    shp = jax.ShapeDtypeStruct((_R, N_MOL // _R), jnp.float32)
    tot, lf, le = pl.pallas_call(
        _fin_body,
        out_shape=(shp, shp, shp),
    )(
        partial[0].reshape(_R, -1),
        partial[1].reshape(_R, -1),
        atomic_subsystem_counts.reshape(_R, -1),
        per_molecule_energy_predict.reshape(_R, -1),
        per_molecule_energy_true.reshape(_R, -1),
    )
    out = (tot.reshape(N_MOL, 1), lf.reshape(N_MOL, 1), le.reshape(N_MOL, 1))
    return out


# speedup vs baseline: 1.8420x; 1.8243x over previous
import jax
import jax.numpy as jnp
from jax import lax
from jax.experimental import pallas as pl
from jax.experimental.pallas import tpu as pltpu
from jax.experimental.pallas import tpu_sc as plsc

N_ATOMS = 1638400
N_MOL = 16384
W_FORCE = 0.999
W_ENERGY = 0.001

NC = 2
NS = 16
LANES = 16

NW = NC * NS
APW = N_ATOMS // NW
ROUNDS = 5
CPT = APW // ROUNDS
VPR = CPT // LANES
MPT = N_MOL // NS


def _vtake(x, idx):
    return x.at[idx].get(mode="promise_in_bounds")


def _sc_body(d_hbm, idx_hbm, out_hbm,
             d_buf, i_buf, acc, red, tmp, slab, sem):
    c = lax.axis_index("c")
    s = lax.axis_index("s")
    wid = c * NS + s
    wbase = wid * APW

    lanes = lax.iota(jnp.int32, LANES)
    zero16 = jnp.zeros((LANES,), jnp.float32)

    def _zero(j, carry):
        acc[pl.ds(j * LANES, LANES)] = zero16
        return carry
    lax.fori_loop(0, N_MOL // LANES, _zero, 0)

    DSZ = 3 * CPT
    ISZ = CPT
    iota_prev = jnp.maximum(lanes - 1, 0)
    iota_next = jnp.minimum(lanes + 1, LANES - 1)
    fifteen = jnp.full((LANES,), LANES - 1, jnp.int32)
    zeros_i = jnp.zeros((LANES,), jnp.int32)
    is_lane0 = lanes == 0

    def _issue(r, b):
        a0 = wbase + r * CPT
        return [
            pltpu.async_copy(
                d_hbm.at[pl.ds(3 * a0, 3 * CPT)],
                d_buf.at[pl.ds(b * DSZ, 3 * CPT)], sem),
            pltpu.async_copy(
                idx_hbm.at[pl.ds(a0, CPT)],
                i_buf.at[pl.ds(b * ISZ, CPT)], sem),
        ]

    def _vreg(m, m_next, dx, dy, dz, carry):
        e = dx * dx + dy * dy + dz * dz
        ctot = jnp.cumsum(e) + carry
        b = m != m_next
        u = jnp.where(b, ctot, 0.0)
        w = plsc.cummax(u)
        wsh = _vtake(w, iota_prev, )
        pb = jnp.where(is_lane0, 0.0, wsh)
        plsc.addupdate_scatter(acc, [m], ctot - pb, mask=b)
        t15 = _vtake(ctot, fifteen, )
        w15 = _vtake(w, fifteen, )
        return t15 - w15

    carry = zero16
    pend = _issue(0, 0)
    for r in range(ROUNDS):
        b = r % 2
        for d in pend:
            d.wait()
        pend = _issue(r + 1, (r + 1) % 2) if r + 1 < ROUNDS else []
        ib = b * ISZ
        db = b * DSZ

        def _body(j, carry, _ib=ib, _db=db):
            a = j * LANES
            g = jax.lax.shift_right_logical(a, 7)
            pd = _db + a + g * 256
            m = i_buf[pl.ds(_ib + a, LANES)]
            m_next = i_buf[pl.ds(_ib + a + 1, LANES)]
            dx = d_buf[pl.ds(pd, LANES)]
            dy = d_buf[pl.ds(pd + 128, LANES)]
            dz = d_buf[pl.ds(pd + 256, LANES)]
            return _vreg(m, m_next, dx, dy, dz, carry)

        carry = plsc.parallel_loop(
            0, VPR - 1, carry=carry)(_body)

        a = (VPR - 1) * LANES
        pd = db + a + ((a >> 7) << 8)
        m = i_buf[pl.ds(ib + a, LANES)]
        m_sh = _vtake(m, iota_next, )
        if r + 1 < ROUNDS:
            for d in pend:
                d.wait()
            pend = []
            mn0 = i_buf[pl.ds(((r + 1) % 2) * ISZ, LANES)]
            mn0s = _vtake(mn0, zeros_i, )
            m_next = jnp.where(lanes == LANES - 1, mn0s, m_sh)
        else:
            m_next = m_sh
        dx = d_buf[pl.ds(pd, LANES)]
        dy = d_buf[pl.ds(pd + 128, LANES)]
        dz = d_buf[pl.ds(pd + 256, LANES)]
        carry = _vreg(m, m_next, dx, dy, dz, carry)
        m_last = _vtake(m, fifteen, )

    plsc.addupdate_scatter(acc, [m_last], carry, mask=is_lane0)

    pltpu.sync_copy(acc, slab.at[pl.ds(s * N_MOL, N_MOL)])
    plsc.subcore_barrier()
    pltpu.sync_copy(slab.at[pl.ds(s * MPT, MPT)], red)
    for j in range(1, NS):
        pltpu.sync_copy(slab.at[pl.ds(j * N_MOL + s * MPT, MPT)], tmp)

        def _acc(q, carry):
            sl = pl.ds(q * LANES, LANES)
            red[sl] = red[sl] + tmp[sl]
            return carry
        lax.fori_loop(0, MPT // LANES, _acc, 0)
    pltpu.sync_copy(red, out_hbm.at[c, pl.ds(s * MPT, MPT)])


_sc_partial = pl.kernel(
    _sc_body,
    out_type=jax.ShapeDtypeStruct((NC, N_MOL), jnp.float32),
    mesh=plsc.VectorSubcoreMesh(core_axis_name="c", subcore_axis_name="s"),
    scratch_types=[
        pltpu.VMEM((2 * 3 * CPT,), jnp.float32),
        pltpu.VMEM((2 * CPT,), jnp.int32),
        pltpu.VMEM((N_MOL,), jnp.float32),
        pltpu.VMEM((MPT,), jnp.float32),
        pltpu.VMEM((MPT,), jnp.float32),
        pltpu.VMEM_SHARED((NS * N_MOL,), jnp.float32),
        pltpu.SemaphoreType.DMA,
    ],
    compiler_params=pltpu.CompilerParams(needs_layout_passes=False),
)


def _fin_body(pm0, pm1, cnt, ep, et, tot, lf, le):
    counts = cnt[...].astype(jnp.float32)
    force = (pm0[...] + pm1[...]) / (3.0 * counts)
    d = ep[...] - et[...]
    energy = (d * d) / counts
    tot[...] = W_FORCE * force + W_ENERGY * energy
    lf[...] = force
    le[...] = energy


_R = 128


def kernel(per_atom_force_predict, per_atom_force_true,
           per_molecule_energy_predict, per_molecule_energy_true,
           atomic_subsystem_indices, atomic_subsystem_counts):
    d_flat = ((per_atom_force_predict - per_atom_force_true)
              .T.reshape(3, N_ATOMS // 128, 128)
              .transpose(1, 0, 2).reshape(-1))
    partial = _sc_partial(d_flat, atomic_subsystem_indices)

    shp = jax.ShapeDtypeStruct((_R, N_MOL // _R), jnp.float32)
    tot, lf, le = pl.pallas_call(
        _fin_body,
        out_shape=(shp, shp, shp),
    )(
        partial[0].reshape(_R, -1),
        partial[1].reshape(_R, -1),
        atomic_subsystem_counts.reshape(_R, -1),
        per_molecule_energy_predict.reshape(_R, -1),
        per_molecule_energy_true.reshape(_R, -1),
    )
    out = (tot.reshape(N_MOL, 1), lf.reshape(N_MOL, 1), le.reshape(N_MOL, 1))
    return out
